# Initial kernel scaffold; baseline (speedup 1.0000x reference)
import jax
import jax.numpy as jnp
from jax.experimental import pallas as pl


def kernel(x, table, pos_emb):
    raise NotImplementedError("write your pallas kernel here")


# speedup vs baseline: 1.0761x; 1.0761x over previous
import functools

import jax
import jax.numpy as jnp
from jax import lax
from jax.experimental import pallas as pl
from jax.experimental.pallas import tpu as pltpu
from jax.experimental.pallas import tpu_sc as plsc

VOCAB = 100000
D = 128
BATCH = 4
SEQ = 2048
TOTAL = BATCH * SEQ

_info = plsc.get_sparse_core_info()
NC = _info.num_cores
NS = _info.num_subcores
NW = NC * NS
L = _info.num_lanes

ROWS_PER_W = TOTAL // NW
VECS_PER_ROW = D // L


def _sc_body(x_hbm, table_hbm, pos_hbm, out_hbm, idx_v, rows_v, pos_v, sem):
    wid = lax.axis_index("s") * NC + lax.axis_index("c")
    base = wid * ROWS_PER_W
    pos_base = lax.rem(base, SEQ)

    pltpu.sync_copy(x_hbm.at[pl.ds(base, ROWS_PER_W)], idx_v)
    gather = pltpu.async_copy(table_hbm.at[idx_v], rows_v, sem)
    pltpu.sync_copy(pos_hbm.at[pl.ds(pos_base, ROWS_PER_W)], pos_v)
    gather.wait()

    def add_row(r):
        for c in range(VECS_PER_ROW):
            sl = pl.ds(c * L, L)
            plsc.addupdate(rows_v.at[r, sl], pos_v[r, sl])

    pl.loop(0, ROWS_PER_W)(add_row)

    pltpu.sync_copy(rows_v, out_hbm.at[pl.ds(base, ROWS_PER_W)])


@jax.jit
def _sc_call(x_flat, table, pos_emb):
    mesh = plsc.VectorSubcoreMesh(core_axis_name="c", subcore_axis_name="s")
    kfn = functools.partial(
        pl.kernel,
        mesh=mesh,
        out_type=jax.ShapeDtypeStruct((TOTAL, D), jnp.float32),
        scratch_types=[
            pltpu.VMEM((ROWS_PER_W,), jnp.int32),
            pltpu.VMEM((ROWS_PER_W, D), jnp.float32),
            pltpu.VMEM((ROWS_PER_W, D), jnp.float32),
            pltpu.SemaphoreType.DMA,
        ],
    )(_sc_body)
    return kfn(x_flat, table, pos_emb)


def kernel(x, table, pos_emb):
    x_flat = x.reshape(TOTAL).astype(jnp.int32)
    out = _sc_call(x_flat, table, pos_emb)
    return out.reshape(BATCH, SEQ, D)


# speedup vs baseline: 1.1367x; 1.0564x over previous
import functools

import jax
import jax.numpy as jnp
from jax import lax
from jax.experimental import pallas as pl
from jax.experimental.pallas import tpu as pltpu
from jax.experimental.pallas import tpu_sc as plsc

VOCAB = 100000
D = 128
BATCH = 4
SEQ = 2048
TOTAL = BATCH * SEQ

_info = plsc.get_sparse_core_info()
NC = _info.num_cores
NS = _info.num_subcores
NW = NC * NS

ROWS_PER_W = TOTAL // NW


def _sc_body(x_hbm, table_hbm, pos_hbm, out_hbm, idx_v, rows_v, sem):
    wid = lax.axis_index("s") * NC + lax.axis_index("c")
    base = wid * ROWS_PER_W
    pos_base = lax.rem(base, SEQ)

    pltpu.sync_copy(x_hbm.at[pl.ds(base, ROWS_PER_W)], idx_v)
    pltpu.sync_copy(pos_hbm.at[pl.ds(pos_base, ROWS_PER_W)], rows_v)
    pltpu.async_copy(table_hbm.at[idx_v], rows_v, sem, add=True).wait()
    pltpu.sync_copy(rows_v, out_hbm.at[pl.ds(base, ROWS_PER_W)])


@jax.jit
def _sc_call(x_flat, table, pos_emb):
    mesh = plsc.VectorSubcoreMesh(core_axis_name="c", subcore_axis_name="s")
    kfn = functools.partial(
        pl.kernel,
        mesh=mesh,
        out_type=jax.ShapeDtypeStruct((TOTAL, D), jnp.float32),
        scratch_types=[
            pltpu.VMEM((ROWS_PER_W,), jnp.int32),
            pltpu.VMEM((ROWS_PER_W, D), jnp.float32),
            pltpu.SemaphoreType.DMA,
        ],
    )(_sc_body)
    return kfn(x_flat, table, pos_emb)


def kernel(x, table, pos_emb):
    x_flat = x.reshape(TOTAL).astype(jnp.int32)
    out = _sc_call(x_flat, table, pos_emb)
    return out.reshape(BATCH, SEQ, D)
